# baseline (speedup 1.0000x reference)
import functools

import jax
import jax.numpy as jnp
from jax import lax
from jax.experimental import pallas as pl
from jax.experimental.pallas import tpu as pltpu
from jax.experimental.pallas import tpu_sc as plsc

_C = 1
_O = 64
_L = 50
_N = 100000
_F = 128

_SCAT = 128
_LOAD = 256
_NLOAD = _N // _LOAD
_REM = _N - _NLOAD * _LOAD
_QROWS = 128
_TROWS = _REM - _QROWS


def _seg_sum_sc(x, labels):
    info = plsc.get_sparse_core_info()
    nc, ns = info.num_cores, info.num_subcores
    nw = nc * ns

    nslots = (_NLOAD + nw - 1) // nw
    last_cut = _NLOAD - (nslots - 1) * nw

    mesh = plsc.VectorSubcoreMesh(core_axis_name="c", subcore_axis_name="s")

    @functools.partial(
        pl.kernel,
        out_type=jax.ShapeDtypeStruct((nc, _L, _F), jnp.float32),
        mesh=mesh,
        scratch_types=[
            pltpu.VMEM((2, _LOAD, _F), jnp.float32),
            pltpu.VMEM((2, 2, _SCAT), jnp.int32),
            pltpu.VMEM((_QROWS, _F), jnp.float32),
            pltpu.VMEM((_QROWS,), jnp.int32),
            pltpu.VMEM((_TROWS, _F), jnp.float32),
            pltpu.VMEM((_TROWS,), jnp.int32),
            pltpu.VMEM((_L, _F), jnp.float32),
            pltpu.VMEM_SHARED((_L, _F), jnp.float32),
            pltpu.SemaphoreType.DMA,
            pltpu.SemaphoreType.DMA,
            pltpu.SemaphoreType.DMA,
            pltpu.SemaphoreType.DMA,
            pltpu.SemaphoreType.DMA,
            pltpu.SemaphoreType.DMA,
            pltpu.SemaphoreType.DMA,
        ],
    )
    def seg_kernel(x_hbm, lbl_hbm, out_hbm, x_v, lbl_v, xq_v, lblq_v, xt_v,
                   lblt_v, zero_v, acc_sh, sx0, sx1, sl0, sl1, sc0, sc1, sq):
        cid = lax.axis_index("c")
        sid = lax.axis_index("s")
        wid = sid * nc + cid
        sx = (sx0, sx1)
        sl = (sl0, sl1)
        sc = (sc0, sc1)

        def mk_loads(i, b):
            m = wid + i * nw
            row0 = pl.multiple_of(m * _LOAD, _LOAD)
            dl0 = pltpu.make_async_copy(
                lbl_hbm.at[pl.ds(row0, _SCAT)], lbl_v.at[b, 0], sl[b])
            dl1 = pltpu.make_async_copy(
                lbl_hbm.at[pl.ds(row0 + _SCAT, _SCAT)], lbl_v.at[b, 1], sl[b])
            dx = pltpu.make_async_copy(
                x_hbm.at[pl.ds(row0, _LOAD), :], x_v.at[b], sx[b])
            return dl0, dl1, dx

        def start_loads(i, b):
            for d in mk_loads(i, b):
                d.start()

        def mk_scats(b):
            return [
                pltpu.make_async_copy(
                    x_v.at[b, pl.ds(j * _SCAT, _SCAT), :],
                    acc_sh.at[lbl_v.at[b, j]], sc[b])
                for j in range(_LOAD // _SCAT)
            ]

        start_loads(0, 0)
        q0 = _NLOAD * _LOAD
        dql = pltpu.make_async_copy(lbl_hbm.at[pl.ds(q0, _QROWS)], lblq_v, sq)
        dqx = pltpu.make_async_copy(x_hbm.at[pl.ds(q0, _QROWS), :], xq_v, sq)
        t0 = q0 + _QROWS
        dtl = pltpu.make_async_copy(lbl_hbm.at[pl.ds(t0, _TROWS)], lblt_v, sq)
        dtx = pltpu.make_async_copy(x_hbm.at[pl.ds(t0, _TROWS), :], xt_v, sq)

        @pl.when(wid == nw - 2)
        def _():
            dql.start()
            dqx.start()

        @pl.when(wid == nw - 1)
        def _():
            dtl.start()
            dtx.start()

        @pl.when(sid == 0)
        def _():
            @pl.loop(0, _L)
            def _(l):
                for j in range(_F // 16):
                    zero_v[l, pl.ds(j * 16, 16)] = jnp.zeros((16,), jnp.float32)
            pltpu.sync_copy(zero_v, acc_sh)

        plsc.subcore_barrier()

        for i in range(nslots):
            b = i & 1

            def body(i=i, b=b):
                if i >= 1:
                    for d in mk_scats(1 - b):
                        d.wait()
                if i + 1 < nslots - 1:
                    start_loads(i + 1, 1 - b)
                elif i + 1 == nslots - 1:
                    @pl.when(wid < last_cut)
                    def _():
                        start_loads(i + 1, 1 - b)
                for d in mk_loads(i, b):
                    d.wait()
                for d in mk_scats(b):
                    d.start(add=True)

            if i < nslots - 1:
                body()
            else:
                pl.when(wid < last_cut)(body)

        last_b = (nslots - 1) & 1

        @pl.when(wid < last_cut)
        def _():
            for d in mk_scats(last_b):
                d.wait()

        @pl.when(jnp.logical_not(wid < last_cut))
        def _():
            for d in mk_scats(1 - last_b):
                d.wait()

        @pl.when(wid == nw - 2)
        def _():
            dql.wait()
            dqx.wait()
            pltpu.sync_copy(xq_v, acc_sh.at[lblq_v], add=True)

        @pl.when(wid == nw - 1)
        def _():
            dtl.wait()
            dtx.wait()
            pltpu.sync_copy(xt_v, acc_sh.at[lblt_v], add=True)

        plsc.subcore_barrier()

        @pl.when(sid == 0)
        def _():
            pltpu.sync_copy(acc_sh, out_hbm.at[cid])

    return seg_kernel(x, labels)


def _combine_tc(partials, w2, b):
    def tc_body(p_ref, w_ref, b_ref, o_ref):
        seg = p_ref[0] + p_ref[1]
        o_ref[...] = (
            jax.lax.dot(w_ref[...], seg, preferred_element_type=jnp.float32)
            + b_ref[0]
        )

    return pl.pallas_call(
        tc_body,
        out_shape=jax.ShapeDtypeStruct((_O, _F), jnp.float32),
    )(partials, w2, b)


def kernel(x, node_labels, Param_W, Param_b):
    labels = node_labels.astype(jnp.int32)
    partials = _seg_sum_sc(x, labels)
    w2 = Param_W.reshape(_O, _L)
    out = _combine_tc(partials, w2, Param_b)
    return out.reshape(_C, _O, _F)
